# Initial kernel scaffold; baseline (speedup 1.0000x reference)
import jax
import jax.numpy as jnp
from jax import lax
from jax.experimental import pallas as pl
from jax.experimental.pallas import tpu as pltpu
from jax.experimental.pallas import tpu_sc as plsc

N_USERS = 50000
N_ITEMS = 50000
N_NODES = N_USERS + N_ITEMS
N_EDGES = 1600000
D = 32
BATCH = 16384

NC = 2
NS = 16
SUB = 128
CHUNK_ROWS = 16
CHUNK = SUB * CHUNK_ROWS
E_PAD = ((N_EDGES + NS * CHUNK - 1) // (NS * CHUNK)) * (NS * CHUNK)
ROWS = E_PAD // SUB
ROWS_PER_TILE = ROWS // NS
CHUNKS_PER_TILE = ROWS_PER_TILE // CHUNK_ROWS

HALF = N_NODES // NC
DUMMY = HALF
ACC_ROWS = HALF + 8
STRIPE = HALF // NS

_mesh = plsc.VectorSubcoreMesh(
    core_axis_name="c", subcore_axis_name="s", num_cores=NC, num_subcores=NS
)


def _layer_body(emb_in, src2d, dst2d, val2d, zeros, emb_out,
                src_v, dst_v, val_v, rows_v, acc_sh, sem):
    c = lax.axis_index("c")
    s = lax.axis_index("s")
    c0 = c * HALF

    pltpu.sync_copy(zeros, acc_sh.at[pl.ds(s * STRIPE, STRIPE)])
    plsc.subcore_barrier()

    def chunk_body(i, carry):
        off = s * ROWS_PER_TILE + i * CHUNK_ROWS
        pltpu.async_copy(src2d.at[pl.ds(off, CHUNK_ROWS)], src_v, sem).wait()
        pltpu.async_copy(dst2d.at[pl.ds(off, CHUNK_ROWS)], dst_v, sem).wait()
        pltpu.async_copy(val2d.at[pl.ds(off, CHUNK_ROWS)], val_v, sem).wait()

        for j in range(CHUNK_ROWS):
            for k in range(SUB // 16):
                d = dst_v[j, pl.ds(k * 16, 16)]
                loc = d - c0
                ok = (d >= c0) & (loc < HALF)
                dst_v[j, pl.ds(k * 16, 16)] = jnp.where(ok, loc, DUMMY)

        for j in range(CHUNK_ROWS):
            pltpu.async_copy(emb_in.at[src_v.at[j]], rows_v, sem).wait()

            def scale_body(e, _):
                v = val_v[j, e]
                rows_v[e, pl.ds(0, 16)] = rows_v[e, pl.ds(0, 16)] * v
                rows_v[e, pl.ds(16, 16)] = rows_v[e, pl.ds(16, 16)] * v
                return 0

            lax.fori_loop(0, SUB, scale_body, 0)
            pltpu.async_copy(rows_v, acc_sh.at[dst_v.at[j]], sem, add=True).wait()
        return carry

    lax.fori_loop(0, CHUNKS_PER_TILE, chunk_body, 0)

    plsc.subcore_barrier()
    pltpu.sync_copy(acc_sh.at[pl.ds(s * STRIPE, STRIPE)],
                    emb_out.at[pl.ds(c0 + s * STRIPE, STRIPE)])


_layer = pl.kernel(
    _layer_body,
    out_type=jax.ShapeDtypeStruct((N_NODES, D), jnp.float32),
    mesh=_mesh,
    scratch_types=[
        pltpu.VMEM((CHUNK_ROWS, SUB), jnp.int32),
        pltpu.VMEM((CHUNK_ROWS, SUB), jnp.int32),
        pltpu.VMEM((CHUNK_ROWS, SUB), jnp.float32),
        pltpu.VMEM((SUB, D), jnp.float32),
        pltpu.VMEM_SHARED((ACC_ROWS, D), jnp.float32),
        pltpu.SemaphoreType.DMA,
    ],
)

B_PER_W = BATCH // (NC * NS)
BROWS_PER_W = B_PER_W // SUB


def _final_body(emb0, emb1, emb2, emb3, users2d, pos2d,
                ue, pe, uf, pf, idx_v, acc_v, sem):
    c = lax.axis_index("c")
    s = lax.axis_index("s")
    wid = s * NC + c
    row0 = wid * BROWS_PER_W
    base = wid * B_PER_W

    def lookup(idx2d, offset, out_raw, out_final):
        pltpu.async_copy(idx2d.at[pl.ds(row0, BROWS_PER_W)], idx_v, sem).wait()
        if offset:
            for j in range(BROWS_PER_W):
                for k in range(SUB // 16):
                    idx_v[j, pl.ds(k * 16, 16)] = (
                        idx_v[j, pl.ds(k * 16, 16)] + offset)
        for j in range(BROWS_PER_W):
            pltpu.async_copy(emb0.at[idx_v.at[j]],
                             acc_v.at[pl.ds(j * SUB, SUB)], sem).wait()
        pltpu.sync_copy(acc_v, out_raw.at[pl.ds(base, B_PER_W)])
        for emb in (emb1, emb2, emb3):
            for j in range(BROWS_PER_W):
                pltpu.async_copy(emb.at[idx_v.at[j]],
                                 acc_v.at[pl.ds(j * SUB, SUB)], sem,
                                 add=True).wait()

        def scale_body(i, _):
            acc_v[i, pl.ds(0, 16)] = acc_v[i, pl.ds(0, 16)] * 0.25
            acc_v[i, pl.ds(16, 16)] = acc_v[i, pl.ds(16, 16)] * 0.25
            return 0

        lax.fori_loop(0, B_PER_W, scale_body, 0)
        pltpu.sync_copy(acc_v, out_final.at[pl.ds(base, B_PER_W)])

    lookup(users2d, 0, ue, uf)
    lookup(pos2d, N_USERS, pe, pf)


_final = pl.kernel(
    _final_body,
    out_type=(
        jax.ShapeDtypeStruct((BATCH, D), jnp.float32),
        jax.ShapeDtypeStruct((BATCH, D), jnp.float32),
        jax.ShapeDtypeStruct((BATCH, D), jnp.float32),
        jax.ShapeDtypeStruct((BATCH, D), jnp.float32),
    ),
    mesh=_mesh,
    scratch_types=[
        pltpu.VMEM((BROWS_PER_W, SUB), jnp.int32),
        pltpu.VMEM((B_PER_W, D), jnp.float32),
        pltpu.SemaphoreType.DMA,
    ],
)


def kernel(user_table, item_table, edge_val, edge_src, edge_dst, users, pos_items):
    emb0 = jnp.concatenate([user_table, item_table], axis=0)
    pad = E_PAD - N_EDGES
    src2d = jnp.concatenate(
        [edge_src.astype(jnp.int32), jnp.zeros((pad,), jnp.int32)]
    ).reshape(ROWS, SUB)
    dst2d = jnp.concatenate(
        [edge_dst.astype(jnp.int32), jnp.zeros((pad,), jnp.int32)]
    ).reshape(ROWS, SUB)
    val2d = jnp.concatenate(
        [edge_val, jnp.zeros((pad,), jnp.float32)]
    ).reshape(ROWS, SUB)
    zeros = jnp.zeros((STRIPE, D), jnp.float32)

    e1 = _layer(emb0, src2d, dst2d, val2d, zeros)
    e2 = _layer(e1, src2d, dst2d, val2d, zeros)
    e3 = _layer(e2, src2d, dst2d, val2d, zeros)

    users2d = users.astype(jnp.int32).reshape(BATCH // SUB, SUB)
    pos2d = pos_items.astype(jnp.int32).reshape(BATCH // SUB, SUB)
    return _final(emb0, e1, e2, e3, users2d, pos2d)


# speedup vs baseline: 7.3546x; 7.3546x over previous
import jax
import jax.numpy as jnp
from jax import lax
from jax.experimental import pallas as pl
from jax.experimental.pallas import tpu as pltpu
from jax.experimental.pallas import tpu_sc as plsc

N_USERS = 50000
N_ITEMS = 50000
N_NODES = N_USERS + N_ITEMS
N_EDGES = 1600000
D = 32
BATCH = 16384

NC = 2
NS = 16
SUB = 128
CHUNK_ROWS = 16
CHUNK = SUB * CHUNK_ROWS
E_PAD = ((N_EDGES + NS * CHUNK - 1) // (NS * CHUNK)) * (NS * CHUNK)
ROWS = E_PAD // SUB
ROWS_PER_TILE = ROWS // NS
CHUNKS_PER_TILE = ROWS_PER_TILE // CHUNK_ROWS

HALF = N_NODES // NC
DUMMY = HALF
ACC_ROWS = HALF + 8
STRIPE = 3128
LAST_STRIPE = HALF - (NS - 1) * STRIPE

_mesh = plsc.VectorSubcoreMesh(
    core_axis_name="c", subcore_axis_name="s", num_cores=NC, num_subcores=NS
)


def _layer_body(emb_in, src2d, dst2d, val2d, zeros, emb_out,
                src_v, dst_v, val_v, rows_v, acc_sh, sem):
    c = lax.axis_index("c")
    s = lax.axis_index("s")
    c0 = c * HALF

    @pl.when(s < NS - 1)
    def _zero_full():
        pltpu.sync_copy(zeros, acc_sh.at[pl.ds(s * STRIPE, STRIPE)])

    @pl.when(s == NS - 1)
    def _zero_last():
        pltpu.sync_copy(zeros.at[pl.ds(0, LAST_STRIPE)],
                        acc_sh.at[pl.ds(s * STRIPE, LAST_STRIPE)])

    plsc.subcore_barrier()

    def chunk_body(i, carry):
        off = s * ROWS_PER_TILE + i * CHUNK_ROWS
        pltpu.async_copy(src2d.at[pl.ds(off, CHUNK_ROWS)], src_v, sem).wait()
        pltpu.async_copy(dst2d.at[pl.ds(off, CHUNK_ROWS)], dst_v, sem).wait()
        pltpu.async_copy(val2d.at[pl.ds(off, CHUNK_ROWS)], val_v, sem).wait()

        for j in range(CHUNK_ROWS):
            for k in range(SUB // 16):
                d = dst_v[j, pl.ds(k * 16, 16)]
                loc = d - c0
                ok = (d >= c0) & (loc < HALF)
                dst_v[j, pl.ds(k * 16, 16)] = jnp.where(ok, loc, DUMMY)

        for j in range(CHUNK_ROWS):
            pltpu.async_copy(emb_in.at[src_v.at[j]], rows_v, sem).wait()

            def scale_body(g, _):
                vv = val_v[j, pl.ds(g * 16, 16)]
                for i in range(16):
                    e = g * 16 + i
                    vi = vv[i]
                    rows_v[e, pl.ds(0, 16)] = rows_v[e, pl.ds(0, 16)] * vi
                    rows_v[e, pl.ds(16, 16)] = rows_v[e, pl.ds(16, 16)] * vi
                return 0

            lax.fori_loop(0, SUB // 16, scale_body, 0)
            pltpu.async_copy(rows_v, acc_sh.at[dst_v.at[j]], sem, add=True).wait()
        return carry

    lax.fori_loop(0, CHUNKS_PER_TILE, chunk_body, 0)

    plsc.subcore_barrier()

    @pl.when(s < NS - 1)
    def _wb_full():
        pltpu.sync_copy(acc_sh.at[pl.ds(s * STRIPE, STRIPE)],
                        emb_out.at[pl.ds(c0 + s * STRIPE, STRIPE)])

    @pl.when(s == NS - 1)
    def _wb_last():
        pltpu.sync_copy(acc_sh.at[pl.ds(s * STRIPE, LAST_STRIPE)],
                        emb_out.at[pl.ds(c0 + s * STRIPE, LAST_STRIPE)])


_params = pltpu.CompilerParams(use_tc_tiling_on_sc=False)

_layer = pl.kernel(
    _layer_body,
    out_type=jax.ShapeDtypeStruct((N_NODES, D), jnp.float32),
    mesh=_mesh,
    compiler_params=_params,
    scratch_types=[
        pltpu.VMEM((CHUNK_ROWS, SUB), jnp.int32),
        pltpu.VMEM((CHUNK_ROWS, SUB), jnp.int32),
        pltpu.VMEM((CHUNK_ROWS, SUB), jnp.float32),
        pltpu.VMEM((SUB, D), jnp.float32),
        pltpu.VMEM_SHARED((ACC_ROWS, D), jnp.float32),
        pltpu.SemaphoreType.DMA,
    ],
)

B_PER_W = BATCH // (NC * NS)
BROWS_PER_W = B_PER_W // SUB


def _final_body(emb0, emb1, emb2, emb3, users2d, pos2d,
                ue, pe, uf, pf, idx_v, acc_v, sem):
    c = lax.axis_index("c")
    s = lax.axis_index("s")
    wid = s * NC + c
    row0 = wid * BROWS_PER_W
    base = wid * B_PER_W

    def lookup(idx2d, offset, out_raw, out_final):
        pltpu.async_copy(idx2d.at[pl.ds(row0, BROWS_PER_W)], idx_v, sem).wait()
        if offset:
            for j in range(BROWS_PER_W):
                for k in range(SUB // 16):
                    idx_v[j, pl.ds(k * 16, 16)] = (
                        idx_v[j, pl.ds(k * 16, 16)] + offset)
        for j in range(BROWS_PER_W):
            pltpu.async_copy(emb0.at[idx_v.at[j]],
                             acc_v.at[pl.ds(j * SUB, SUB)], sem).wait()
        pltpu.sync_copy(acc_v, out_raw.at[pl.ds(base, B_PER_W)])
        for emb in (emb1, emb2, emb3):
            for j in range(BROWS_PER_W):
                pltpu.async_copy(emb.at[idx_v.at[j]],
                                 acc_v.at[pl.ds(j * SUB, SUB)], sem,
                                 add=True).wait()

        def scale_body(i, _):
            acc_v[i, pl.ds(0, 16)] = acc_v[i, pl.ds(0, 16)] * 0.25
            acc_v[i, pl.ds(16, 16)] = acc_v[i, pl.ds(16, 16)] * 0.25
            return 0

        lax.fori_loop(0, B_PER_W, scale_body, 0)
        pltpu.sync_copy(acc_v, out_final.at[pl.ds(base, B_PER_W)])

    lookup(users2d, 0, ue, uf)
    lookup(pos2d, N_USERS, pe, pf)


_final = pl.kernel(
    _final_body,
    out_type=(
        jax.ShapeDtypeStruct((BATCH, D), jnp.float32),
        jax.ShapeDtypeStruct((BATCH, D), jnp.float32),
        jax.ShapeDtypeStruct((BATCH, D), jnp.float32),
        jax.ShapeDtypeStruct((BATCH, D), jnp.float32),
    ),
    mesh=_mesh,
    compiler_params=_params,
    scratch_types=[
        pltpu.VMEM((BROWS_PER_W, SUB), jnp.int32),
        pltpu.VMEM((B_PER_W, D), jnp.float32),
        pltpu.SemaphoreType.DMA,
    ],
)


def kernel(user_table, item_table, edge_val, edge_src, edge_dst, users, pos_items):
    emb0 = jnp.concatenate([user_table, item_table], axis=0)
    pad = E_PAD - N_EDGES
    src2d = jnp.concatenate(
        [edge_src.astype(jnp.int32), jnp.zeros((pad,), jnp.int32)]
    ).reshape(ROWS, SUB)
    dst2d = jnp.concatenate(
        [edge_dst.astype(jnp.int32), jnp.zeros((pad,), jnp.int32)]
    ).reshape(ROWS, SUB)
    val2d = jnp.concatenate(
        [edge_val, jnp.zeros((pad,), jnp.float32)]
    ).reshape(ROWS, SUB)
    zeros = jnp.zeros((STRIPE, D), jnp.float32)

    e1 = _layer(emb0, src2d, dst2d, val2d, zeros)
    e2 = _layer(e1, src2d, dst2d, val2d, zeros)
    e3 = _layer(e2, src2d, dst2d, val2d, zeros)

    users2d = users.astype(jnp.int32).reshape(BATCH // SUB, SUB)
    pos2d = pos_items.astype(jnp.int32).reshape(BATCH // SUB, SUB)
    return _final(emb0, e1, e2, e3, users2d, pos2d)
